# baseline (speedup 1.0000x reference)
import jax
import jax.numpy as jnp
from jax.experimental import pallas as pl
from jax.experimental.pallas import tpu as pltpu

BINS = 255
LOW = -20.0
HIGH = 20.0

BLKR = 8


def _twohot_block(value_ref, out_ref):
    x = value_ref[...]
    v = jnp.clip(jnp.sign(x) * jnp.log1p(jnp.abs(x)), LOW, HIGH)
    pos = (v - LOW) / (HIGH - LOW) * (BINS - 1)
    iota = jax.lax.broadcasted_iota(jnp.int32, out_ref.shape, 2).astype(jnp.float32)
    out_ref[...] = jnp.maximum(1.0 - jnp.abs(pos[:, :, None] - iota), 0.0)


def kernel(value):
    rows, cols = value.shape
    out = pl.pallas_call(
        _twohot_block,
        grid=(rows // BLKR,),
        in_specs=[pl.BlockSpec((BLKR, cols), lambda i: (i, 0))],
        out_specs=pl.BlockSpec((BLKR, cols, BINS), lambda i: (i, 0, 0)),
        out_shape=jax.ShapeDtypeStruct((rows, cols, BINS), jnp.float32),
        compiler_params=pltpu.CompilerParams(
            dimension_semantics=("parallel",),
        ),
    )(value)
    return out
